# baseline (speedup 1.0000x reference)
import functools

import jax
import jax.numpy as jnp
from jax import lax
from jax.experimental import pallas as pl
from jax.experimental.pallas import tpu as pltpu
from jax.experimental.pallas import tpu_sc as plsc

Q = 128
D = 512
NKEY = 200000
KTOP = 100
BK = 8192
NBLK = 25
KPAD = NBLK * BK
CHUNK = 128
NCHUNK = KPAD // CHUNK
NREAL_CHUNK = 1563
CPB = BK // CHUNK
SUB = 32
NSUB = KPAD // SUB
NEG = -3e38

NWORKER = 32
QPW = Q // NWORKER
NCAND = 128
W = 240


def _score_body(q_ref, k_ref, s_ref, m_ref, m32_ref):
    i = pl.program_id(0)
    s = lax.dot_general(q_ref[...], k_ref[...],
                        (((1,), (1,)), ((), ())),
                        preferred_element_type=jnp.float32)
    base = i * BK
    col = base + lax.broadcasted_iota(jnp.int32, (Q, BK), 1)
    s = jnp.where(col < NKEY, s, NEG)
    s3 = s.reshape(Q, CPB, CHUNK)
    s_ref[...] = s3
    m32 = jnp.max(s.reshape(Q, BK // SUB, SUB), axis=2)
    m32_ref[...] = m32
    m_ref[...] = jnp.max(m32.reshape(Q, CPB, CHUNK // SUB), axis=2)[None]


def _scores_and_chunkmax(queries, keys):
    return pl.pallas_call(
        _score_body,
        grid=(NBLK,),
        in_specs=[
            pl.BlockSpec((Q, D), lambda i: (0, 0)),
            pl.BlockSpec((BK, D), lambda i: (i, 0)),
        ],
        out_specs=[
            pl.BlockSpec((Q, CPB, CHUNK), lambda i: (0, i, 0)),
            pl.BlockSpec((1, Q, CPB), lambda i: (i, 0, 0)),
            pl.BlockSpec((Q, BK // SUB), lambda i: (0, i)),
        ],
        out_shape=[
            jax.ShapeDtypeStruct((Q, NCHUNK, CHUNK), jnp.float32),
            jax.ShapeDtypeStruct((NBLK, Q, CPB), jnp.float32),
            jax.ShapeDtypeStruct((Q, NSUB), jnp.float32),
        ],
    )(queries, keys)


def _tau_body(m_ref, tau_ref):
    cm = m_ref[...]
    bits = pltpu.bitcast(cm, jnp.int32)
    u = bits ^ (lax.shift_right_arithmetic(bits, 31) & jnp.int32(0x7FFFFFFF))

    def count_ge(c):
        return jnp.sum((u >= c).astype(jnp.int32), axis=1, keepdims=True)

    t0 = jnp.full((Q, 1), jnp.int32(-2147483648))
    t0 = jnp.where(count_ge(jnp.zeros((Q, 1), jnp.int32)) >= KTOP,
                   jnp.zeros((Q, 1), jnp.int32), t0)

    def step(j, t):
        c = t | lax.shift_left(jnp.int32(1), 30 - j)
        return jnp.where(count_ge(c) >= KTOP, c, t)

    t = lax.fori_loop(0, 31, step, t0)
    fbits = jnp.where(t >= 0, t, t ^ jnp.int32(0x7FFFFFFF))
    tau = pltpu.bitcast(fbits, jnp.float32)
    tau_ref[...] = jnp.broadcast_to(tau, (Q, 16))


def _tau(cmax):
    return pl.pallas_call(
        _tau_body,
        out_shape=jax.ShapeDtypeStruct((Q, 16), jnp.float32),
    )(cmax)


IDW = NCAND + 16
CW = W + 16
L2CAP = 128
L2W = L2CAP + 16


def _sc_body(cmax_hbm, tau_hbm, cmax32_hbm, srows_hbm, outv_hbm, outi_hbm,
             cmax_v, tau_v, cmax32_v, idlist_v, l2list_v, rows_v,
             candv_v, candi_v, sem):
    cid = lax.axis_index("c")
    sid = lax.axis_index("s")
    wid = sid * 2 + cid
    q0 = wid * QPW
    iota16 = lax.iota(jnp.int32, 16)
    pltpu.sync_copy(cmax_hbm.at[pl.ds(q0 * NCHUNK, QPW * NCHUNK)], cmax_v)
    pltpu.sync_copy(tau_hbm.at[pl.ds(q0 * 16, QPW * 16)], tau_v)
    cp32 = pltpu.async_copy(cmax32_hbm.at[pl.ds(q0 * NSUB, QPW * NSUB)],
                            cmax32_v, sem)

    for i in range(QPW):
        pad_row = (q0 + i) * NCHUNK + NREAL_CHUNK
        for j in range(IDW // 16):
            idlist_v[pl.ds(i * IDW + j * 16, 16)] = jnp.broadcast_to(
                pad_row, (16,)).astype(jnp.int32)

    cnts = []
    copies = []
    for i in range(QPW):
        tau = tau_v[pl.ds(i * 16, 16)]
        base_id = (q0 + i) * NCHUNK
        qoff = i * IDW

        def scan_body(j, cnt_vec, i=i, tau=tau, base_id=base_id, qoff=qoff):
            vec = cmax_v[pl.ds(i * NCHUNK + j * 16, 16)]
            m = vec >= tau
            cum = plsc.cumsum(m.astype(jnp.int32))
            pos = jnp.where(m, jnp.minimum(cnt_vec + cum - 1, NCAND),
                            NCAND) + qoff
            ids = base_id + j * 16 + iota16
            plsc.store_scatter(idlist_v, [pos], ids)
            return cnt_vec + plsc.all_reduce_population_count(m)

        cnt_vec = lax.fori_loop(0, NCHUNK // 16, scan_body,
                                jnp.zeros((16,), jnp.int32), unroll=2)
        cnts.append(jnp.max(jnp.minimum(cnt_vec, NCAND)))
        copies.append(pltpu.async_copy(
            srows_hbm.at[idlist_v.at[pl.ds(qoff, NCAND)]],
            rows_v.at[pl.ds(i * NCAND, NCAND)], sem))

    for i in range(QPW):
        for j in range(CW // 16):
            o = i * CW + j * 16
            candv_v[pl.ds(o, 16)] = jnp.full((16,), NEG, jnp.float32)
            candi_v[pl.ds(o, 16)] = (1 << 30) + o + iota16

    cp32.wait()

    l2cnts = []
    for i in range(QPW):
        tau = tau_v[pl.ds(i * 16, 16)]
        base_id = (q0 + i) * NCHUNK
        l2off = i * L2W
        sub4 = iota16 & 3
        row4 = lax.shift_right_logical(iota16, 2)

        def f1_body(rb, lc, i=i, tau=tau, base_id=base_id, l2off=l2off,
                    sub4=sub4, row4=row4):
            r_vec = rb * 4 + row4
            cid_v = plsc.load_gather(idlist_v, [i * IDW + r_vec])
            mx = plsc.load_gather(
                cmax32_v,
                [i * NSUB + (cid_v - base_id) * (CHUNK // SUB) + sub4])
            m = mx >= tau
            cum = plsc.cumsum(m.astype(jnp.int32))
            pos = jnp.where(m, jnp.minimum(lc + cum - 1, L2CAP),
                            L2CAP) + l2off
            plsc.store_scatter(l2list_v, [pos],
                               lax.shift_left(r_vec, 2) | sub4)
            return lc + plsc.all_reduce_population_count(m)

        nb = lax.shift_right_logical(cnts[i] + 3, 2)
        lc_vec = lax.fori_loop(0, nb, f1_body, jnp.zeros((16,), jnp.int32))
        l2cnts.append(jnp.max(jnp.minimum(lc_vec, L2CAP)))

    for i in range(QPW):
        copies[i].wait()
        tau = tau_v[pl.ds(i * 16, 16)]
        base_id = (q0 + i) * NCHUNK
        coff = i * CW
        l2off = i * L2W

        def f2_body(e, cc, i=i, tau=tau, base_id=base_id, coff=coff,
                    l2off=l2off):
            ent = plsc.load_gather(
                l2list_v, [jnp.broadcast_to(l2off + e, (16,)
                                            ).astype(jnp.int32)])
            rowpos = lax.shift_right_logical(ent, 2)
            sub = ent & 3
            cid_v = plsc.load_gather(idlist_v, [i * IDW + rowpos])
            kbase = (cid_v - base_id) * CHUNK + sub * SUB
            rglob = i * NCAND + rowpos
            for k in range(SUB // 16):
                col = sub * SUB + k * 16 + iota16
                vals = plsc.load_gather(rows_v, [rglob, col])
                m = vals >= tau
                cum = plsc.cumsum(m.astype(jnp.int32))
                pos = jnp.where(m, jnp.minimum(cc + cum - 1, W), W) + coff
                plsc.store_scatter(candv_v, [pos], vals)
                plsc.store_scatter(candi_v, [pos], kbase + k * 16 + iota16)
                cc = cc + plsc.all_reduce_population_count(m)
            return cc

        lax.fori_loop(0, l2cnts[i], f2_body, jnp.zeros((16,), jnp.int32))

    pltpu.sync_copy(candv_v, outv_hbm.at[pl.ds(q0 * CW, QPW * CW)])
    pltpu.sync_copy(candi_v, outi_hbm.at[pl.ds(q0 * CW, QPW * CW)])


def _sc_select(cmax_flat, tau_flat, cmax32_flat, srows):
    mesh = plsc.VectorSubcoreMesh(core_axis_name="c", subcore_axis_name="s",
                                  num_cores=2, num_subcores=16)
    f = functools.partial(
        pl.kernel,
        out_type=[jax.ShapeDtypeStruct((Q * CW,), jnp.float32),
                  jax.ShapeDtypeStruct((Q * CW,), jnp.int32)],
        mesh=mesh,
        compiler_params=pltpu.CompilerParams(needs_layout_passes=False),
        scratch_types=[
            pltpu.VMEM((QPW * NCHUNK,), jnp.float32),
            pltpu.VMEM((QPW * 16,), jnp.float32),
            pltpu.VMEM((QPW * NSUB,), jnp.float32),
            pltpu.VMEM((QPW * IDW,), jnp.int32),
            pltpu.VMEM((QPW * L2W,), jnp.int32),
            pltpu.VMEM((QPW * NCAND, CHUNK), jnp.float32),
            pltpu.VMEM((QPW * CW,), jnp.float32),
            pltpu.VMEM((QPW * CW,), jnp.int32),
            pltpu.SemaphoreType.DMA,
        ],
    )(_sc_body)
    return f(cmax_flat, tau_flat, cmax32_flat, srows)


def _topk_body(v_ref, i_ref, ov_ref, oi_ref):
    vals0 = v_ref[...]
    idxs = i_ref[...]
    lane = lax.broadcasted_iota(jnp.int32, (Q, 128), 1)

    def step(j, carry):
        vals, accv, acci = carry
        m = jnp.max(vals, axis=1, keepdims=True)
        sel = jnp.min(jnp.where(vals == m, idxs, jnp.int32(2147483647)),
                      axis=1, keepdims=True)
        accv = jnp.where(lane == j, m, accv)
        acci = jnp.where(lane == j, sel, acci)
        vals = jnp.where(idxs == sel, NEG, vals)
        return vals, accv, acci

    _, accv, acci = lax.fori_loop(
        0, KTOP, step,
        (vals0, jnp.full((Q, 128), NEG, jnp.float32),
         jnp.zeros((Q, 128), jnp.int32)))
    ov_ref[...] = accv[:, :KTOP]
    oi_ref[...] = acci[:, :KTOP]


def _topk(candv, candi):
    return pl.pallas_call(
        _topk_body,
        out_shape=[jax.ShapeDtypeStruct((Q, KTOP), jnp.float32),
                   jax.ShapeDtypeStruct((Q, KTOP), jnp.int32)],
    )(candv, candi)


def kernel(queries, keys):
    scores3, cmax3, cmax32 = _scores_and_chunkmax(queries, keys)
    cmax = jnp.transpose(cmax3, (1, 0, 2)).reshape(Q, NCHUNK)
    tau = _tau(cmax32)
    candv, candi = _sc_select(cmax.reshape(-1), tau.reshape(-1),
                              cmax32.reshape(-1),
                              scores3.reshape(Q * NCHUNK, CHUNK))
    return _topk(candv.reshape(Q, CW), candi.reshape(Q, CW))


# speedup vs baseline: 1.8937x; 1.8937x over previous
import functools

import jax
import jax.numpy as jnp
from jax import lax
from jax.experimental import pallas as pl
from jax.experimental.pallas import tpu as pltpu
from jax.experimental.pallas import tpu_sc as plsc

Q = 128
D = 512
NKEY = 200000
KTOP = 100
BK = 8192
NBLK = 25
KPAD = NBLK * BK
CHUNK = 128
NCHUNK = KPAD // CHUNK
NREAL_CHUNK = 1563
CPB = BK // CHUNK
SUB = 32
NSUB = KPAD // SUB
NEG = -3e38

NWORKER = 32
QPW = Q // NWORKER
NCAND = 128
W = 240


def _score_body(q_ref, k_ref, s_ref, m_ref, m32_ref):
    i = pl.program_id(0)
    s = lax.dot_general(q_ref[...], k_ref[...],
                        (((1,), (1,)), ((), ())),
                        preferred_element_type=jnp.float32)
    base = i * BK
    col = base + lax.broadcasted_iota(jnp.int32, (Q, BK), 1)
    s = jnp.where(col < NKEY, s, NEG)
    s3 = s.reshape(Q, CPB, CHUNK)
    s_ref[...] = s3
    m_ref[...] = jnp.max(s3, axis=2)[None]
    parts = [jnp.max(s3[:, :, o * SUB:(o + 1) * SUB], axis=2)
             for o in range(CHUNK // SUB)]
    m32_ref[...] = jnp.concatenate(parts, axis=1)[None]


def _scores_and_chunkmax(queries, keys):
    return pl.pallas_call(
        _score_body,
        grid=(NBLK,),
        in_specs=[
            pl.BlockSpec((Q, D), lambda i: (0, 0)),
            pl.BlockSpec((BK, D), lambda i: (i, 0)),
        ],
        out_specs=[
            pl.BlockSpec((Q, CPB, CHUNK), lambda i: (0, i, 0)),
            pl.BlockSpec((1, Q, CPB), lambda i: (i, 0, 0)),
            pl.BlockSpec((1, Q, BK // SUB), lambda i: (i, 0, 0)),
        ],
        out_shape=[
            jax.ShapeDtypeStruct((Q, NCHUNK, CHUNK), jnp.float32),
            jax.ShapeDtypeStruct((NBLK, Q, CPB), jnp.float32),
            jax.ShapeDtypeStruct((NBLK, Q, BK // SUB), jnp.float32),
        ],
    )(queries, keys)


def _tau_body(m_ref, tau_ref):
    cm = m_ref[...]
    bits = pltpu.bitcast(cm, jnp.int32)
    u = bits ^ (lax.shift_right_arithmetic(bits, 31) & jnp.int32(0x7FFFFFFF))

    def count_ge(c):
        ge = (u >= c[None]).astype(jnp.int32)
        return jnp.sum(jnp.sum(ge, axis=2, keepdims=True), axis=0)

    t0 = jnp.full((Q, 1), jnp.int32(-2147483648))
    t0 = jnp.where(count_ge(jnp.zeros((Q, 1), jnp.int32)) >= KTOP,
                   jnp.zeros((Q, 1), jnp.int32), t0)

    def step(j, t):
        c = t | lax.shift_left(jnp.int32(1), 30 - j)
        return jnp.where(count_ge(c) >= KTOP, c, t)

    t = lax.fori_loop(0, 31, step, t0)
    fbits = jnp.where(t >= 0, t, t ^ jnp.int32(0x7FFFFFFF))
    tau = pltpu.bitcast(fbits, jnp.float32)
    tau_ref[...] = jnp.broadcast_to(tau, (Q, 16))


def _tau(cmax):
    return pl.pallas_call(
        _tau_body,
        out_shape=jax.ShapeDtypeStruct((Q, 16), jnp.float32),
    )(cmax)


IDW = NCAND + 16
CW = W + 16
L2CAP = 128
L2W = L2CAP + 16


def _sc_body(cmax_hbm, tau_hbm, cmax32_hbm, srows_hbm, outv_hbm, outi_hbm,
             cmax_v, tau_v, cmax32_v, idlist_v, l2list_v, rows_v,
             candv_v, candi_v, sem):
    cid = lax.axis_index("c")
    sid = lax.axis_index("s")
    wid = sid * 2 + cid
    q0 = wid * QPW
    iota16 = lax.iota(jnp.int32, 16)
    pltpu.sync_copy(cmax_hbm.at[pl.ds(q0 * NCHUNK, QPW * NCHUNK)], cmax_v)
    pltpu.sync_copy(tau_hbm.at[pl.ds(q0 * 16, QPW * 16)], tau_v)
    cp32 = pltpu.async_copy(cmax32_hbm.at[pl.ds(q0 * NSUB, QPW * NSUB)],
                            cmax32_v, sem)

    for i in range(QPW):
        pad_row = (q0 + i) * NCHUNK + NREAL_CHUNK
        for j in range(IDW // 16):
            idlist_v[pl.ds(i * IDW + j * 16, 16)] = jnp.broadcast_to(
                pad_row, (16,)).astype(jnp.int32)

    cnts = []
    copies = []
    for i in range(QPW):
        tau = tau_v[pl.ds(i * 16, 16)]
        base_id = (q0 + i) * NCHUNK
        qoff = i * IDW

        def scan_body(j, cnt_vec, i=i, tau=tau, base_id=base_id, qoff=qoff):
            vec = cmax_v[pl.ds(i * NCHUNK + j * 16, 16)]
            m = vec >= tau
            cum = plsc.cumsum(m.astype(jnp.int32))
            pos = jnp.where(m, jnp.minimum(cnt_vec + cum - 1, NCAND),
                            NCAND) + qoff
            ids = base_id + j * 16 + iota16
            plsc.store_scatter(idlist_v, [pos], ids)
            return cnt_vec + plsc.all_reduce_population_count(m)

        cnt_vec = lax.fori_loop(0, NCHUNK // 16, scan_body,
                                jnp.zeros((16,), jnp.int32), unroll=2)
        cnts.append(jnp.max(jnp.minimum(cnt_vec, NCAND)))
        copies.append(pltpu.async_copy(
            srows_hbm.at[idlist_v.at[pl.ds(qoff, NCAND)]],
            rows_v.at[pl.ds(i * NCAND, NCAND)], sem))

    for i in range(QPW):
        for j in range(CW // 16):
            o = i * CW + j * 16
            candv_v[pl.ds(o, 16)] = jnp.full((16,), NEG, jnp.float32)
            candi_v[pl.ds(o, 16)] = (1 << 30) + o + iota16

    cp32.wait()

    l2cnts = []
    for i in range(QPW):
        tau = tau_v[pl.ds(i * 16, 16)]
        base_id = (q0 + i) * NCHUNK
        l2off = i * L2W
        sub4 = iota16 & 3
        row4 = lax.shift_right_logical(iota16, 2)

        def f1_body(rb, lc, i=i, tau=tau, base_id=base_id, l2off=l2off,
                    sub4=sub4, row4=row4):
            r_vec = rb * 4 + row4
            cid_v = plsc.load_gather(idlist_v, [i * IDW + r_vec])
            chunk = cid_v - base_id
            b = lax.div(chunk, CPB)
            col = b * (BK // SUB) + sub4 * CPB + (chunk - b * CPB)
            mx = plsc.load_gather(cmax32_v, [i * NSUB + col])
            m = mx >= tau
            cum = plsc.cumsum(m.astype(jnp.int32))
            pos = jnp.where(m, jnp.minimum(lc + cum - 1, L2CAP),
                            L2CAP) + l2off
            plsc.store_scatter(l2list_v, [pos],
                               lax.shift_left(r_vec, 2) | sub4)
            return lc + plsc.all_reduce_population_count(m)

        nb = lax.shift_right_logical(cnts[i] + 3, 2)
        lc_vec = lax.fori_loop(0, nb, f1_body, jnp.zeros((16,), jnp.int32))
        l2cnts.append(jnp.max(jnp.minimum(lc_vec, L2CAP)))

    for i in range(QPW):
        copies[i].wait()
        tau = tau_v[pl.ds(i * 16, 16)]
        base_id = (q0 + i) * NCHUNK
        coff = i * CW
        l2off = i * L2W

        def f2_body(e, cc, i=i, tau=tau, base_id=base_id, coff=coff,
                    l2off=l2off):
            ent = plsc.load_gather(
                l2list_v, [jnp.broadcast_to(l2off + e, (16,)
                                            ).astype(jnp.int32)])
            rowpos = lax.shift_right_logical(ent, 2)
            sub = ent & 3
            cid_v = plsc.load_gather(idlist_v, [i * IDW + rowpos])
            kbase = (cid_v - base_id) * CHUNK + sub * SUB
            rglob = i * NCAND + rowpos
            for k in range(SUB // 16):
                col = sub * SUB + k * 16 + iota16
                vals = plsc.load_gather(rows_v, [rglob, col])
                m = vals >= tau
                cum = plsc.cumsum(m.astype(jnp.int32))
                pos = jnp.where(m, jnp.minimum(cc + cum - 1, W), W) + coff
                plsc.store_scatter(candv_v, [pos], vals)
                plsc.store_scatter(candi_v, [pos], kbase + k * 16 + iota16)
                cc = cc + plsc.all_reduce_population_count(m)
            return cc

        lax.fori_loop(0, l2cnts[i], f2_body, jnp.zeros((16,), jnp.int32))

    pltpu.sync_copy(candv_v, outv_hbm.at[pl.ds(q0 * CW, QPW * CW)])
    pltpu.sync_copy(candi_v, outi_hbm.at[pl.ds(q0 * CW, QPW * CW)])


def _sc_select(cmax_flat, tau_flat, cmax32_flat, srows):
    mesh = plsc.VectorSubcoreMesh(core_axis_name="c", subcore_axis_name="s",
                                  num_cores=2, num_subcores=16)
    f = functools.partial(
        pl.kernel,
        out_type=[jax.ShapeDtypeStruct((Q * CW,), jnp.float32),
                  jax.ShapeDtypeStruct((Q * CW,), jnp.int32)],
        mesh=mesh,
        compiler_params=pltpu.CompilerParams(needs_layout_passes=False),
        scratch_types=[
            pltpu.VMEM((QPW * NCHUNK,), jnp.float32),
            pltpu.VMEM((QPW * 16,), jnp.float32),
            pltpu.VMEM((QPW * NSUB,), jnp.float32),
            pltpu.VMEM((QPW * IDW,), jnp.int32),
            pltpu.VMEM((QPW * L2W,), jnp.int32),
            pltpu.VMEM((QPW * NCAND, CHUNK), jnp.float32),
            pltpu.VMEM((QPW * CW,), jnp.float32),
            pltpu.VMEM((QPW * CW,), jnp.int32),
            pltpu.SemaphoreType.DMA,
        ],
    )(_sc_body)
    return f(cmax_flat, tau_flat, cmax32_flat, srows)


def _topk_body(v_ref, i_ref, ov_ref, oi_ref):
    vals0 = v_ref[...]
    idxs = i_ref[...]
    lane = lax.broadcasted_iota(jnp.int32, (Q, 128), 1)

    def step(j, carry):
        vals, accv, acci = carry
        m = jnp.max(vals, axis=1, keepdims=True)
        sel = jnp.min(jnp.where(vals == m, idxs, jnp.int32(2147483647)),
                      axis=1, keepdims=True)
        accv = jnp.where(lane == j, m, accv)
        acci = jnp.where(lane == j, sel, acci)
        vals = jnp.where(idxs == sel, NEG, vals)
        return vals, accv, acci

    _, accv, acci = lax.fori_loop(
        0, KTOP, step,
        (vals0, jnp.full((Q, 128), NEG, jnp.float32),
         jnp.zeros((Q, 128), jnp.int32)))
    ov_ref[...] = accv[:, :KTOP]
    oi_ref[...] = acci[:, :KTOP]


def _topk(candv, candi):
    return pl.pallas_call(
        _topk_body,
        out_shape=[jax.ShapeDtypeStruct((Q, KTOP), jnp.float32),
                   jax.ShapeDtypeStruct((Q, KTOP), jnp.int32)],
    )(candv, candi)


def kernel(queries, keys):
    scores3, cmax3, cmax32 = _scores_and_chunkmax(queries, keys)
    cmax = jnp.transpose(cmax3, (1, 0, 2)).reshape(Q, NCHUNK)
    tau = _tau(cmax32)
    cmax32_q = jnp.transpose(cmax32, (1, 0, 2)).reshape(Q, NSUB)
    candv, candi = _sc_select(cmax.reshape(-1), tau.reshape(-1),
                              cmax32_q.reshape(-1),
                              scores3.reshape(Q * NCHUNK, CHUNK))
    return _topk(candv.reshape(Q, CW), candi.reshape(Q, CW))


# speedup vs baseline: 2.1187x; 1.1188x over previous
import functools

import jax
import jax.numpy as jnp
from jax import lax
from jax.experimental import pallas as pl
from jax.experimental.pallas import tpu as pltpu
from jax.experimental.pallas import tpu_sc as plsc

Q = 128
D = 512
NKEY = 200000
KTOP = 100
BK = 8192
NBLK = 25
KPAD = NBLK * BK
CHUNK = 128
NCHUNK = KPAD // CHUNK
NREAL_CHUNK = 1563
CPB = BK // CHUNK
NEG = -3e38

NWORKER = 32
QPW = Q // NWORKER
NCAND = 128
W = 240


def _score_body(q_ref, k_ref, s_ref, m_ref):
    i = pl.program_id(0)
    s = lax.dot_general(q_ref[...], k_ref[...],
                        (((1,), (1,)), ((), ())),
                        preferred_element_type=jnp.float32)
    base = i * BK
    col = base + lax.broadcasted_iota(jnp.int32, (Q, BK), 1)
    s = jnp.where(col < NKEY, s, NEG)
    s3 = s.reshape(Q, CPB, CHUNK)
    s_ref[...] = s3
    m_ref[...] = jnp.max(s3, axis=2)[None]


def _scores_and_chunkmax(queries, keys):
    return pl.pallas_call(
        _score_body,
        grid=(NBLK,),
        in_specs=[
            pl.BlockSpec((Q, D), lambda i: (0, 0)),
            pl.BlockSpec((BK, D), lambda i: (i, 0)),
        ],
        out_specs=[
            pl.BlockSpec((Q, CPB, CHUNK), lambda i: (0, i, 0)),
            pl.BlockSpec((1, Q, CPB), lambda i: (i, 0, 0)),
        ],
        out_shape=[
            jax.ShapeDtypeStruct((Q, NCHUNK, CHUNK), jnp.float32),
            jax.ShapeDtypeStruct((NBLK, Q, CPB), jnp.float32),
        ],
    )(queries, keys)


def _tau_body(m_ref, tau_ref):
    cm = m_ref[...]
    bits = pltpu.bitcast(cm, jnp.int32)
    u = bits ^ (lax.shift_right_arithmetic(bits, 31) & jnp.int32(0x7FFFFFFF))

    def count_ge(c):
        return jnp.sum((u >= c).astype(jnp.int32), axis=1, keepdims=True)

    t0 = jnp.full((Q, 1), jnp.int32(-2147483648))
    t0 = jnp.where(count_ge(jnp.zeros((Q, 1), jnp.int32)) >= KTOP,
                   jnp.zeros((Q, 1), jnp.int32), t0)

    def step(j, t):
        c = t | lax.shift_left(jnp.int32(1), 30 - j)
        return jnp.where(count_ge(c) >= KTOP, c, t)

    t = lax.fori_loop(0, 31, step, t0)
    fbits = jnp.where(t >= 0, t, t ^ jnp.int32(0x7FFFFFFF))
    tau = pltpu.bitcast(fbits, jnp.float32)
    tau_ref[...] = jnp.broadcast_to(tau, (Q, 16))


def _tau(cmax):
    return pl.pallas_call(
        _tau_body,
        out_shape=jax.ShapeDtypeStruct((Q, 16), jnp.float32),
    )(cmax)


IDW = NCAND + 16
CW = W + 16


def _sc_body(cmax_hbm, tau_hbm, srows_hbm, outv_hbm, outi_hbm,
             cmax_v, tau_v, idlist_v, rows_v, candv_v, candi_v, sem):
    cid = lax.axis_index("c")
    sid = lax.axis_index("s")
    wid = sid * 2 + cid
    q0 = wid * QPW
    iota16 = lax.iota(jnp.int32, 16)
    pltpu.sync_copy(cmax_hbm.at[pl.ds(q0 * NCHUNK, QPW * NCHUNK)], cmax_v)
    pltpu.sync_copy(tau_hbm.at[pl.ds(q0 * 16, QPW * 16)], tau_v)

    for i in range(QPW):
        pad_row = (q0 + i) * NCHUNK + NREAL_CHUNK
        for j in range(IDW // 16):
            idlist_v[pl.ds(i * IDW + j * 16, 16)] = jnp.broadcast_to(
                pad_row, (16,)).astype(jnp.int32)

    cnts = []
    copies = []
    for i in range(QPW):
        tau = tau_v[pl.ds(i * 16, 16)]
        base_id = (q0 + i) * NCHUNK
        qoff = i * IDW

        def scan_body(j, cnt_vec, i=i, tau=tau, base_id=base_id, qoff=qoff):
            vec = cmax_v[pl.ds(i * NCHUNK + j * 16, 16)]
            m = vec >= tau
            cum = plsc.cumsum(m.astype(jnp.int32))
            pos = jnp.where(m, jnp.minimum(cnt_vec + cum - 1, NCAND),
                            NCAND) + qoff
            ids = base_id + j * 16 + iota16
            plsc.store_scatter(idlist_v, [pos], ids)
            return cnt_vec + plsc.all_reduce_population_count(m)

        cnt_vec = lax.fori_loop(0, NCHUNK // 16, scan_body,
                                jnp.zeros((16,), jnp.int32), unroll=2)
        cnts.append(jnp.max(jnp.minimum(cnt_vec, NCAND)))
        copies.append(pltpu.async_copy(
            srows_hbm.at[idlist_v.at[pl.ds(qoff, NCAND)]],
            rows_v.at[pl.ds(i * NCAND, NCAND)], sem))

    for i in range(QPW):
        for j in range(CW // 16):
            o = i * CW + j * 16
            candv_v[pl.ds(o, 16)] = jnp.full((16,), NEG, jnp.float32)
            candi_v[pl.ds(o, 16)] = (1 << 30) + o + iota16

    for i in range(QPW):
        copies[i].wait()
        tau = tau_v[pl.ds(i * 16, 16)]
        base_id = (q0 + i) * NCHUNK
        coff = i * CW

        def filt_row(r, cc, i=i, tau=tau, base_id=base_id, coff=coff):
            rr = i * NCAND + r
            rsplat = jnp.broadcast_to(rr, (16,)).astype(jnp.int32)
            cvec = plsc.load_gather(idlist_v,
                                    [jnp.broadcast_to(i * IDW + r, (16,)
                                                      ).astype(jnp.int32)])
            cbase = (cvec - base_id) * CHUNK
            for k in range(8):
                vals = plsc.load_gather(rows_v, [rsplat, k * 16 + iota16])
                m = vals >= tau
                cum = plsc.cumsum(m.astype(jnp.int32))
                pos = jnp.where(m, jnp.minimum(cc + cum - 1, W), W) + coff
                plsc.store_scatter(candv_v, [pos], vals)
                plsc.store_scatter(candi_v, [pos], cbase + k * 16 + iota16)
                cc = cc + plsc.all_reduce_population_count(m)
            return cc

        lax.fori_loop(0, cnts[i], filt_row, jnp.zeros((16,), jnp.int32))

    pltpu.sync_copy(candv_v, outv_hbm.at[pl.ds(q0 * CW, QPW * CW)])
    pltpu.sync_copy(candi_v, outi_hbm.at[pl.ds(q0 * CW, QPW * CW)])


def _sc_select(cmax_flat, tau_flat, srows):
    mesh = plsc.VectorSubcoreMesh(core_axis_name="c", subcore_axis_name="s",
                                  num_cores=2, num_subcores=16)
    f = functools.partial(
        pl.kernel,
        out_type=[jax.ShapeDtypeStruct((Q * CW,), jnp.float32),
                  jax.ShapeDtypeStruct((Q * CW,), jnp.int32)],
        mesh=mesh,
        compiler_params=pltpu.CompilerParams(needs_layout_passes=False),
        scratch_types=[
            pltpu.VMEM((QPW * NCHUNK,), jnp.float32),
            pltpu.VMEM((QPW * 16,), jnp.float32),
            pltpu.VMEM((QPW * IDW,), jnp.int32),
            pltpu.VMEM((QPW * NCAND, CHUNK), jnp.float32),
            pltpu.VMEM((QPW * CW,), jnp.float32),
            pltpu.VMEM((QPW * CW,), jnp.int32),
            pltpu.SemaphoreType.DMA,
        ],
    )(_sc_body)
    return f(cmax_flat, tau_flat, srows)


def _topk_body(v_ref, i_ref, ov_ref, oi_ref):
    vals0 = v_ref[...]
    idxs = i_ref[...]
    lane = lax.broadcasted_iota(jnp.int32, (Q, 128), 1)

    def step(j, carry):
        vals, accv, acci = carry
        m = jnp.max(vals, axis=1, keepdims=True)
        sel = jnp.min(jnp.where(vals == m, idxs, jnp.int32(2147483647)),
                      axis=1, keepdims=True)
        accv = jnp.where(lane == j, m, accv)
        acci = jnp.where(lane == j, sel, acci)
        vals = jnp.where(idxs == sel, NEG, vals)
        return vals, accv, acci

    _, accv, acci = lax.fori_loop(
        0, KTOP, step,
        (vals0, jnp.full((Q, 128), NEG, jnp.float32),
         jnp.zeros((Q, 128), jnp.int32)))
    ov_ref[...] = accv[:, :KTOP]
    oi_ref[...] = acci[:, :KTOP]


def _topk(candv, candi):
    return pl.pallas_call(
        _topk_body,
        out_shape=[jax.ShapeDtypeStruct((Q, KTOP), jnp.float32),
                   jax.ShapeDtypeStruct((Q, KTOP), jnp.int32)],
    )(candv, candi)


def kernel(queries, keys):
    scores3, cmax3 = _scores_and_chunkmax(queries, keys)
    cmax = jnp.transpose(cmax3, (1, 0, 2)).reshape(Q, NCHUNK)
    tau = _tau(cmax)
    candv, candi = _sc_select(cmax.reshape(-1), tau.reshape(-1),
                              scores3.reshape(Q * NCHUNK, CHUNK))
    return _topk(candv.reshape(Q, CW), candi.reshape(Q, CW))
